# baseline (speedup 1.0000x reference)
import functools

import jax
import jax.numpy as jnp
from jax import lax
from jax.experimental import pallas as pl
from jax.experimental.pallas import tpu as pltpu
from jax.experimental.pallas import tpu_sc as plsc

NC = 2
NS = 16
NW = NC * NS


def _gemm_gid_body(n_nodes, x_ref, w_ref, s_ref, r_ref, o_ref, g_ref):
    o_ref[0] = jnp.dot(x_ref[...], w_ref[0],
                       preferred_element_type=jnp.float32)
    g_ref[...] = r_ref[...] * n_nodes + s_ref[...]


def _relation_gemm_gid(x, weight, src, rel, bn):
    n, d_in = x.shape
    r, _, d_out = weight.shape
    e = src.shape[0]
    nb = r * (n // bn)
    eb = 8
    ew = e // (nb * eb)
    assert eb * ew * nb == e
    s2 = src.reshape(nb * eb, ew)
    r2 = rel.reshape(nb * eb, ew)
    return pl.pallas_call(
        functools.partial(_gemm_gid_body, n),
        grid=(r, n // bn),
        in_specs=[
            pl.BlockSpec((bn, d_in), lambda i, j: (j, 0)),
            pl.BlockSpec((1, d_in, d_out), lambda i, j: (i, 0, 0)),
            pl.BlockSpec((eb, ew), lambda i, j, _nbj=n // bn: (i * _nbj + j, 0)),
            pl.BlockSpec((eb, ew), lambda i, j, _nbj=n // bn: (i * _nbj + j, 0)),
        ],
        out_specs=[
            pl.BlockSpec((1, bn, d_out), lambda i, j: (i, j, 0)),
            pl.BlockSpec((eb, ew), lambda i, j, _nbj=n // bn: (i * _nbj + j, 0)),
        ],
        out_shape=[
            jax.ShapeDtypeStruct((r, n, d_out), jnp.float32),
            jax.ShapeDtypeStruct((nb * eb, ew), jnp.int32),
        ],
    )(x, weight, s2, r2)


def _add_body(p_ref, o_ref):
    o_ref[...] = p_ref[0] + p_ref[1]


def _pair_add(p, bn):
    _, n, d = p.shape
    return pl.pallas_call(
        _add_body,
        grid=(n // bn,),
        in_specs=[pl.BlockSpec((2, bn, d), lambda i: (0, i, 0))],
        out_specs=pl.BlockSpec((bn, d), lambda i: (i, 0)),
        out_shape=jax.ShapeDtypeStruct((n, d), jnp.float32),
    )(p)


def _make_sc_scatter(n_nodes, d, n_edges):
    ept = n_edges // NW
    b = 104
    nch = -(-ept // b)
    nch += (1 - nch) % 2
    ept_p = nch * b
    rpt_a = (n_nodes // NS) & ~7
    rpt_z = n_nodes - rpt_a * (NS - 1)
    assert ept * NW == n_edges and nch % 2 == 1 and nch >= 5
    assert b % 8 == 0 and ept_p % 8 == 0 and rpt_a % 8 == 0

    mesh = plsc.VectorSubcoreMesh(core_axis_name="c", subcore_axis_name="s",
                                  num_cores=NC, num_subcores=NS)

    @functools.partial(
        pl.kernel,
        out_type=jax.ShapeDtypeStruct((NC, n_nodes, d), jnp.float32),
        mesh=mesh,
        scratch_types=[
            pltpu.VMEM((ept_p,), jnp.int32),
            pltpu.VMEM((nch, b), jnp.int32),
            pltpu.VMEM((2, b, d), jnp.float32),
            pltpu.VMEM_SHARED((n_nodes + NS, d), jnp.float32),
            pltpu.SemaphoreType.DMA,
            pltpu.SemaphoreType.DMA,
            pltpu.SemaphoreType.DMA,
            pltpu.SemaphoreType.DMA,
        ],
    )
    def sc_scatter(y_hbm, g_hbm, dst_hbm, zeros_hbm, out_hbm,
                   gid, did2, rows, h_sh, sem_i, sem_d, sem_a, sem_b):
        cid = lax.axis_index("c")
        sid = lax.axis_index("s")
        wid = cid * NS + sid
        base = wid * ept_p

        cp_g = pltpu.async_copy(g_hbm.at[pl.ds(base, ept_p)], gid, sem_i)

        def fill_did(c, carry):
            pltpu.async_copy(dst_hbm.at[pl.ds(base + c * b, b)],
                             did2.at[c], sem_d)
            return carry

        lax.fori_loop(0, nch, fill_did, 0)

        @pl.when(sid < NS - 1)
        def _():
            pltpu.sync_copy(zeros_hbm.at[pl.ds(sid * rpt_a, rpt_a)],
                            h_sh.at[pl.ds(sid * rpt_a, rpt_a)])

        @pl.when(sid == NS - 1)
        def _():
            pltpu.sync_copy(zeros_hbm.at[pl.ds(rpt_a * (NS - 1), rpt_z)],
                            h_sh.at[pl.ds(rpt_a * (NS - 1), rpt_z)])

        cp_g.wait()

        def drain_did(c, carry):
            pltpu.make_async_copy(dst_hbm.at[pl.ds(base, b)],
                                  did2.at[0], sem_d).wait()
            return carry

        lax.fori_loop(0, nch, drain_did, 0)

        plsc.subcore_barrier()

        pltpu.async_copy(y_hbm.at[gid.at[pl.ds(0, b)]], rows.at[0], sem_a)

        def step(t, carry):
            c0 = 2 * t
            c1 = 2 * t + 1
            pltpu.make_async_copy(y_hbm.at[gid.at[pl.ds(c0 * b, b)]],
                                  rows.at[0], sem_a).wait()
            pltpu.async_copy(y_hbm.at[gid.at[pl.ds(c1 * b, b)]],
                             rows.at[1], sem_b)
            pltpu.sync_copy(rows.at[0], h_sh.at[did2.at[c0]], add=True)
            pltpu.make_async_copy(y_hbm.at[gid.at[pl.ds(c1 * b, b)]],
                                  rows.at[1], sem_b).wait()
            pltpu.async_copy(y_hbm.at[gid.at[pl.ds((c1 + 1) * b, b)]],
                             rows.at[0], sem_a)
            pltpu.sync_copy(rows.at[1], h_sh.at[did2.at[c1]], add=True)
            return carry

        lax.fori_loop(0, (nch - 1) // 2, step, 0)
        pltpu.make_async_copy(y_hbm.at[gid.at[pl.ds((nch - 1) * b, b)]],
                              rows.at[0], sem_a).wait()
        pltpu.sync_copy(rows.at[0], h_sh.at[did2.at[nch - 1]], add=True)

        plsc.subcore_barrier()

        @pl.when(sid < NS - 1)
        def _():
            pltpu.sync_copy(h_sh.at[pl.ds(sid * rpt_a, rpt_a)],
                            out_hbm.at[cid, pl.ds(sid * rpt_a, rpt_a)])

        @pl.when(sid == NS - 1)
        def _():
            pltpu.sync_copy(h_sh.at[pl.ds(rpt_a * (NS - 1), rpt_z)],
                            out_hbm.at[cid, pl.ds(rpt_a * (NS - 1), rpt_z)])

    return sc_scatter, ept, ept_p


def _pad_per_tile(a, ept, ept_p, fill):
    if ept_p == ept:
        return a
    pad = jnp.broadcast_to(jnp.asarray(fill, a.dtype).reshape(-1, 1),
                           (NW, ept_p - ept))
    return jnp.concatenate([a.reshape(NW, ept), pad], axis=1).reshape(-1)


def kernel(x, edge_index, rel_type, weight):
    n, _ = x.shape
    r, _, d_out = weight.shape
    e = edge_index.shape[1]
    src = edge_index[0]
    dst = edge_index[1]
    y, g2 = _relation_gemm_gid(x, weight, src, rel_type, 1000)
    y = y.reshape(r * n, d_out)
    g = g2.reshape(e)
    sc, ept, ept_p = _make_sc_scatter(n, d_out, e)
    g_p = _pad_per_tile(g, ept, ept_p, 0)
    dst_p = _pad_per_tile(dst, ept, ept_p, n + jnp.arange(NW) % NS)
    zeros = jnp.zeros((n, d_out), jnp.float32)
    partials = sc(y, g_p, dst_p, zeros)
    return _pair_add(partials, 1000)


# speedup vs baseline: 1.7510x; 1.7510x over previous
import functools

import jax
import jax.numpy as jnp
from jax import lax
from jax.experimental import pallas as pl
from jax.experimental.pallas import tpu as pltpu
from jax.experimental.pallas import tpu_sc as plsc

NC = 2
NS = 16
NW = NC * NS


def _gemm_gid_body(n_nodes, x_ref, w_ref, s_ref, r_ref, o_ref, g_ref):
    o_ref[0] = jnp.dot(x_ref[...], w_ref[0],
                       preferred_element_type=jnp.float32)
    g_ref[...] = r_ref[...] * n_nodes + s_ref[...]


def _relation_gemm_gid(x, weight, src, rel, bn):
    n, d_in = x.shape
    r, _, d_out = weight.shape
    e = src.shape[0]
    nb = r * (n // bn)
    eb = 8
    ew = e // (nb * eb)
    assert eb * ew * nb == e
    s2 = src.reshape(nb * eb, ew)
    r2 = rel.reshape(nb * eb, ew)
    return pl.pallas_call(
        functools.partial(_gemm_gid_body, n),
        grid=(r, n // bn),
        in_specs=[
            pl.BlockSpec((bn, d_in), lambda i, j: (j, 0)),
            pl.BlockSpec((1, d_in, d_out), lambda i, j: (i, 0, 0)),
            pl.BlockSpec((eb, ew), lambda i, j, _nbj=n // bn: (i * _nbj + j, 0)),
            pl.BlockSpec((eb, ew), lambda i, j, _nbj=n // bn: (i * _nbj + j, 0)),
        ],
        out_specs=[
            pl.BlockSpec((1, bn, d_out), lambda i, j: (i, j, 0)),
            pl.BlockSpec((eb, ew), lambda i, j, _nbj=n // bn: (i * _nbj + j, 0)),
        ],
        out_shape=[
            jax.ShapeDtypeStruct((r, n, d_out), jnp.float32),
            jax.ShapeDtypeStruct((nb * eb, ew), jnp.int32),
        ],
    )(x, weight, s2, r2)


def _add_body(p_ref, o_ref):
    o_ref[...] = p_ref[0] + p_ref[1]


def _pair_add(p, bn):
    _, n, d = p.shape
    return pl.pallas_call(
        _add_body,
        grid=(n // bn,),
        in_specs=[pl.BlockSpec((2, bn, d), lambda i: (0, i, 0))],
        out_specs=pl.BlockSpec((bn, d), lambda i: (i, 0)),
        out_shape=jax.ShapeDtypeStruct((n, d), jnp.float32),
    )(p)


def _make_sc_scatter(n_nodes, d, n_edges):
    ept = n_edges // NW
    b = 80
    nch = -(-ept // b)
    nch += (5 - nch) % 6
    ept_p = nch * b
    rpt_a = (n_nodes // NS) & ~7
    rpt_z = n_nodes - rpt_a * (NS - 1)
    assert ept * NW == n_edges and nch % 6 == 5 and nch >= 17
    assert b % 8 == 0 and ept_p % 8 == 0 and rpt_a % 8 == 0

    mesh = plsc.VectorSubcoreMesh(core_axis_name="c", subcore_axis_name="s",
                                  num_cores=NC, num_subcores=NS)

    @functools.partial(
        pl.kernel,
        out_type=jax.ShapeDtypeStruct((NC, n_nodes, d), jnp.float32),
        mesh=mesh,
        scratch_types=[
            pltpu.VMEM((ept_p,), jnp.int32),
            pltpu.VMEM((6, b), jnp.int32),
            pltpu.VMEM((3, b, d), jnp.float32),
            pltpu.VMEM_SHARED((n_nodes + NS, d), jnp.float32),
            pltpu.SemaphoreType.DMA,
            pltpu.SemaphoreType.DMA,
            (pltpu.SemaphoreType.DMA,) * 3,
            (pltpu.SemaphoreType.DMA,) * 3,
        ],
    )
    def sc_scatter(y_hbm, g_hbm, dst_hbm, zeros_hbm, out_hbm,
                   gid, didr, rows, h_sh, sem_i, sem_d, sg, ss):
        cid = lax.axis_index("c")
        sid = lax.axis_index("s")
        wid = cid * NS + sid
        base = wid * ept_p

        cp_g = pltpu.async_copy(g_hbm.at[pl.ds(base, ept_p)], gid, sem_i)

        def fire_d(c, slot):
            pltpu.async_copy(dst_hbm.at[pl.ds(base + c * b, b)],
                             didr.at[slot], sem_d)

        def wait_d():
            pltpu.make_async_copy(dst_hbm.at[pl.ds(base, b)],
                                  didr.at[0], sem_d).wait()

        @pl.when(sid < NS - 1)
        def _():
            pltpu.sync_copy(zeros_hbm.at[pl.ds(sid * rpt_a, rpt_a)],
                            h_sh.at[pl.ds(sid * rpt_a, rpt_a)])

        @pl.when(sid == NS - 1)
        def _():
            pltpu.sync_copy(zeros_hbm.at[pl.ds(rpt_a * (NS - 1), rpt_z)],
                            h_sh.at[pl.ds(rpt_a * (NS - 1), rpt_z)])

        cp_g.wait()

        plsc.subcore_barrier()

        def fire_g(c, slot):
            pltpu.async_copy(y_hbm.at[gid.at[pl.ds(c * b, b)]],
                             rows.at[slot], sg[slot])

        def wait_g(c, slot):
            pltpu.make_async_copy(y_hbm.at[gid.at[pl.ds(c * b, b)]],
                                  rows.at[slot], sg[slot]).wait()

        def fire_s(c, k3, k6):
            pltpu.async_copy(rows.at[k3], h_sh.at[didr.at[k6]],
                             ss[k3], add=True)

        def wait_s(k3, k6):
            pltpu.make_async_copy(rows.at[k3], h_sh.at[didr.at[k6]],
                                  ss[k3]).wait()

        def visit(c, j, first, last):
            k3 = j % 3
            wait_g(c, k3)
            wait_d()
            fire_s(c, k3, j)
            if not (first and j == 0):
                wait_s((j + 2) % 3, (j + 5) % 6)
            if not last or j + 4 < 5:
                fire_d(c + 4, (j + 4) % 6)
            if not last or j + 2 < 5:
                fire_g(c + 2, (j + 2) % 3)

        for j in range(4):
            fire_d(j, j)
        fire_g(0, 0)
        fire_g(1, 1)
        for c in range(6):
            visit(c, c, first=True, last=False)

        def step(t, carry):
            for j in range(6):
                visit(6 * t + j, j, first=False, last=False)
            return carry

        lax.fori_loop(1, (nch - 5) // 6, step, 0)
        for j in range(5):
            visit(nch - 5 + j, (nch - 5 + j) % 6, first=False, last=True)
        wait_s((nch - 1) % 3, (nch - 1) % 6)

        plsc.subcore_barrier()

        @pl.when(sid < NS - 1)
        def _():
            pltpu.sync_copy(h_sh.at[pl.ds(sid * rpt_a, rpt_a)],
                            out_hbm.at[cid, pl.ds(sid * rpt_a, rpt_a)])

        @pl.when(sid == NS - 1)
        def _():
            pltpu.sync_copy(h_sh.at[pl.ds(rpt_a * (NS - 1), rpt_z)],
                            out_hbm.at[cid, pl.ds(rpt_a * (NS - 1), rpt_z)])

    return sc_scatter, ept, ept_p


def _pad_per_tile(a, ept, ept_p, fill):
    if ept_p == ept:
        return a
    pad = jnp.broadcast_to(jnp.asarray(fill, a.dtype).reshape(-1, 1),
                           (NW, ept_p - ept))
    return jnp.concatenate([a.reshape(NW, ept), pad], axis=1).reshape(-1)


def kernel(x, edge_index, rel_type, weight):
    n, _ = x.shape
    r, _, d_out = weight.shape
    e = edge_index.shape[1]
    src = edge_index[0]
    dst = edge_index[1]
    y, g2 = _relation_gemm_gid(x, weight, src, rel_type, 1000)
    y = y.reshape(r * n, d_out)
    g = g2.reshape(e)
    sc, ept, ept_p = _make_sc_scatter(n, d_out, e)
    g_p = _pad_per_tile(g, ept, ept_p, 0)
    dst_p = _pad_per_tile(dst, ept, ept_p, n + jnp.arange(NW) % NS)
    zeros = jnp.zeros((n, d_out), jnp.float32)
    partials = sc(y, g_p, dst_p, zeros)
    return _pair_add(partials, 1000)


# speedup vs baseline: 1.8731x; 1.0697x over previous
import functools

import jax
import jax.numpy as jnp
from jax import lax
from jax.experimental import pallas as pl
from jax.experimental.pallas import tpu as pltpu
from jax.experimental.pallas import tpu_sc as plsc

NC = 2
NS = 16
NW = NC * NS


def _gemm_gid_body(n_nodes, x_ref, w_ref, s_ref, r_ref, o_ref, g_ref):
    o_ref[...] = jnp.dot(x_ref[...], w_ref[0],
                         preferred_element_type=jnp.float32)

    @pl.when(jnp.logical_and(pl.program_id(0) == 0, pl.program_id(1) == 0))
    def _():
        g_ref[...] = r_ref[...] * n_nodes + s_ref[...]


def _relation_gemm_gid(x, weight, src, rel, bn):
    n, d_in = x.shape
    r, _, d_out = weight.shape
    e = src.shape[0]
    nbn = n // bn
    return pl.pallas_call(
        functools.partial(_gemm_gid_body, n),
        grid=(nbn, r),
        in_specs=[
            pl.BlockSpec((bn, d_in), lambda i, j: (i, 0)),
            pl.BlockSpec((1, d_in, d_out), lambda i, j: (j, 0, 0)),
            pl.BlockSpec((e,), lambda i, j: (0,)),
            pl.BlockSpec((e,), lambda i, j: (0,)),
        ],
        out_specs=[
            pl.BlockSpec((bn, d_out), lambda i, j, _nbn=nbn: (j * _nbn + i, 0)),
            pl.BlockSpec((e,), lambda i, j: (0,)),
        ],
        out_shape=[
            jax.ShapeDtypeStruct((r * n, d_out), jnp.float32),
            jax.ShapeDtypeStruct((e,), jnp.int32),
        ],
    )(x, weight, src, rel)


def _add_body(p_ref, o_ref):
    o_ref[...] = p_ref[0] + p_ref[1]


def _pair_add(p, bn):
    _, n, d = p.shape
    return pl.pallas_call(
        _add_body,
        grid=(n // bn,),
        in_specs=[pl.BlockSpec((2, bn, d), lambda i: (0, i, 0))],
        out_specs=pl.BlockSpec((bn, d), lambda i: (i, 0)),
        out_shape=jax.ShapeDtypeStruct((n, d), jnp.float32),
    )(p)


def _make_sc_scatter(n_nodes, d, n_edges):
    ept = n_edges // NW
    b = 80
    nch = -(-ept // b)
    nch += (5 - nch) % 6
    ept_p = nch * b
    rpt_a = (n_nodes // NS) & ~7
    rpt_z = n_nodes - rpt_a * (NS - 1)
    assert ept * NW == n_edges and nch % 6 == 5 and nch >= 17
    assert b % 8 == 0 and ept_p % 8 == 0 and rpt_a % 8 == 0

    mesh = plsc.VectorSubcoreMesh(core_axis_name="c", subcore_axis_name="s",
                                  num_cores=NC, num_subcores=NS)

    @functools.partial(
        pl.kernel,
        out_type=jax.ShapeDtypeStruct((NC, n_nodes, d), jnp.float32),
        mesh=mesh,
        scratch_types=[
            pltpu.VMEM((ept_p,), jnp.int32),
            pltpu.VMEM((6, b), jnp.int32),
            pltpu.VMEM((3, b, d), jnp.float32),
            pltpu.VMEM_SHARED((n_nodes + NS, d), jnp.float32),
            pltpu.SemaphoreType.DMA,
            pltpu.SemaphoreType.DMA,
            (pltpu.SemaphoreType.DMA,) * 3,
            (pltpu.SemaphoreType.DMA,) * 3,
        ],
    )
    def sc_scatter(y_hbm, g_hbm, dst_hbm, zeros_hbm, out_hbm,
                   gid, didr, rows, h_sh, sem_i, sem_d, sg, ss):
        cid = lax.axis_index("c")
        sid = lax.axis_index("s")
        wid = cid * NS + sid
        base = wid * ept_p

        cp_g = pltpu.async_copy(g_hbm.at[pl.ds(base, ept_p)], gid, sem_i)

        def fire_d(c, slot):
            pltpu.async_copy(dst_hbm.at[pl.ds(base + c * b, b)],
                             didr.at[slot], sem_d)

        def wait_d():
            pltpu.make_async_copy(dst_hbm.at[pl.ds(base, b)],
                                  didr.at[0], sem_d).wait()

        @pl.when(sid < NS - 1)
        def _():
            pltpu.sync_copy(zeros_hbm.at[pl.ds(sid * rpt_a, rpt_a)],
                            h_sh.at[pl.ds(sid * rpt_a, rpt_a)])

        @pl.when(sid == NS - 1)
        def _():
            pltpu.sync_copy(zeros_hbm.at[pl.ds(rpt_a * (NS - 1), rpt_z)],
                            h_sh.at[pl.ds(rpt_a * (NS - 1), rpt_z)])

        cp_g.wait()

        plsc.subcore_barrier()

        def fire_g(c, slot):
            pltpu.async_copy(y_hbm.at[gid.at[pl.ds(c * b, b)]],
                             rows.at[slot], sg[slot])

        def wait_g(c, slot):
            pltpu.make_async_copy(y_hbm.at[gid.at[pl.ds(c * b, b)]],
                                  rows.at[slot], sg[slot]).wait()

        def fire_s(c, k3, k6):
            pltpu.async_copy(rows.at[k3], h_sh.at[didr.at[k6]],
                             ss[k3], add=True)

        def wait_s(k3, k6):
            pltpu.make_async_copy(rows.at[k3], h_sh.at[didr.at[k6]],
                                  ss[k3]).wait()

        def visit(c, j, first, last):
            k3 = j % 3
            wait_g(c, k3)
            wait_d()
            fire_s(c, k3, j)
            if not (first and j == 0):
                wait_s((j + 2) % 3, (j + 5) % 6)
            if not last or j + 4 < 5:
                fire_d(c + 4, (j + 4) % 6)
            if not last or j + 2 < 5:
                fire_g(c + 2, (j + 2) % 3)

        for j in range(4):
            fire_d(j, j)
        fire_g(0, 0)
        fire_g(1, 1)
        for c in range(6):
            visit(c, c, first=True, last=False)

        def step(t, carry):
            for j in range(6):
                visit(6 * t + j, j, first=False, last=False)
            return carry

        lax.fori_loop(1, (nch - 5) // 6, step, 0)
        for j in range(5):
            visit(nch - 5 + j, (nch - 5 + j) % 6, first=False, last=True)
        wait_s((nch - 1) % 3, (nch - 1) % 6)

        plsc.subcore_barrier()

        @pl.when(sid < NS - 1)
        def _():
            pltpu.sync_copy(h_sh.at[pl.ds(sid * rpt_a, rpt_a)],
                            out_hbm.at[cid, pl.ds(sid * rpt_a, rpt_a)])

        @pl.when(sid == NS - 1)
        def _():
            pltpu.sync_copy(h_sh.at[pl.ds(rpt_a * (NS - 1), rpt_z)],
                            out_hbm.at[cid, pl.ds(rpt_a * (NS - 1), rpt_z)])

    return sc_scatter, ept, ept_p


def _pad_per_tile(a, ept, ept_p, fill):
    if ept_p == ept:
        return a
    pad = jnp.broadcast_to(jnp.asarray(fill, a.dtype).reshape(-1, 1),
                           (NW, ept_p - ept))
    return jnp.concatenate([a.reshape(NW, ept), pad], axis=1).reshape(-1)


def kernel(x, edge_index, rel_type, weight):
    n, _ = x.shape
    r, _, d_out = weight.shape
    e = edge_index.shape[1]
    src = edge_index[0]
    dst = edge_index[1]
    y, g = _relation_gemm_gid(x, weight, src, rel_type, 1000)
    sc, ept, ept_p = _make_sc_scatter(n, d_out, e)
    g_p = _pad_per_tile(g, ept, ept_p, 0)
    dst_p = _pad_per_tile(dst, ept, ept_p, n + jnp.arange(NW) % NS)
    zeros = jnp.zeros((n, d_out), jnp.float32)
    partials = sc(y, g_p, dst_p, zeros)
    return _pair_add(partials, 2000)


# speedup vs baseline: 1.9692x; 1.0513x over previous
import functools

import jax
import jax.numpy as jnp
from jax import lax
from jax.experimental import pallas as pl
from jax.experimental.pallas import tpu as pltpu
from jax.experimental.pallas import tpu_sc as plsc

NC = 2
NS = 16
NW = NC * NS


def _gemm_gid_body(n_nodes, x_ref, w_ref, ei_ref, r_ref, o_ref, g_ref, d_ref):
    o_ref[...] = jnp.dot(x_ref[...], w_ref[0],
                         preferred_element_type=jnp.float32)

    @pl.when(jnp.logical_and(pl.program_id(0) == 0, pl.program_id(1) == 0))
    def _():
        g_ref[...] = r_ref[...] * n_nodes + ei_ref[0, 0]
        d_ref[...] = ei_ref[1, 0]


def _relation_gemm_gid(x, weight, edge_index, rel, bn):
    n, d_in = x.shape
    r, _, d_out = weight.shape
    e = rel.shape[0]
    ei3 = edge_index.reshape(2, 1, e)
    nbn = n // bn
    return pl.pallas_call(
        functools.partial(_gemm_gid_body, n),
        grid=(nbn, r),
        in_specs=[
            pl.BlockSpec((bn, d_in), lambda i, j: (i, 0)),
            pl.BlockSpec((1, d_in, d_out), lambda i, j: (j, 0, 0)),
            pl.BlockSpec((2, 1, e), lambda i, j: (0, 0, 0)),
            pl.BlockSpec((e,), lambda i, j: (0,)),
        ],
        out_specs=[
            pl.BlockSpec((bn, d_out), lambda i, j, _nbn=nbn: (j * _nbn + i, 0)),
            pl.BlockSpec((e,), lambda i, j: (0,)),
            pl.BlockSpec((e,), lambda i, j: (0,)),
        ],
        out_shape=[
            jax.ShapeDtypeStruct((r * n, d_out), jnp.float32),
            jax.ShapeDtypeStruct((e,), jnp.int32),
            jax.ShapeDtypeStruct((e,), jnp.int32),
        ],
    )(x, weight, ei3, rel)


def _add_body(p_ref, o_ref):
    o_ref[...] = p_ref[0] + p_ref[1]


def _pair_add(p, bn):
    _, n, d = p.shape
    return pl.pallas_call(
        _add_body,
        grid=(n // bn,),
        in_specs=[pl.BlockSpec((2, bn, d), lambda i: (0, i, 0))],
        out_specs=pl.BlockSpec((bn, d), lambda i: (i, 0)),
        out_shape=jax.ShapeDtypeStruct((n, d), jnp.float32),
    )(p)


def _make_sc_scatter(n_nodes, d, n_edges):
    ept = n_edges // NW
    b = 80
    nch = -(-ept // b)
    nch += (5 - nch) % 6
    ept_p = nch * b
    rpt_a = (n_nodes // NS) & ~7
    rpt_z = n_nodes - rpt_a * (NS - 1)
    assert ept * NW == n_edges and nch % 6 == 5 and nch >= 17
    assert b % 8 == 0 and ept_p % 8 == 0 and rpt_a % 8 == 0

    mesh = plsc.VectorSubcoreMesh(core_axis_name="c", subcore_axis_name="s",
                                  num_cores=NC, num_subcores=NS)

    @functools.partial(
        pl.kernel,
        out_type=jax.ShapeDtypeStruct((NC, n_nodes, d), jnp.float32),
        mesh=mesh,
        scratch_types=[
            pltpu.VMEM((ept_p,), jnp.int32),
            pltpu.VMEM((6, b), jnp.int32),
            pltpu.VMEM((3, b, d), jnp.float32),
            pltpu.VMEM_SHARED((n_nodes + NS, d), jnp.float32),
            pltpu.SemaphoreType.DMA,
            pltpu.SemaphoreType.DMA,
            (pltpu.SemaphoreType.DMA,) * 3,
            (pltpu.SemaphoreType.DMA,) * 3,
        ],
    )
    def sc_scatter(y_hbm, g_hbm, dst_hbm, zeros_hbm, out_hbm,
                   gid, didr, rows, h_sh, sem_i, sem_d, sg, ss):
        cid = lax.axis_index("c")
        sid = lax.axis_index("s")
        wid = cid * NS + sid
        base = wid * ept_p

        cp_g = pltpu.async_copy(g_hbm.at[pl.ds(base, ept_p)], gid, sem_i)

        def fire_d(c, slot):
            pltpu.async_copy(dst_hbm.at[pl.ds(base + c * b, b)],
                             didr.at[slot], sem_d)

        def wait_d():
            pltpu.make_async_copy(dst_hbm.at[pl.ds(base, b)],
                                  didr.at[0], sem_d).wait()

        @pl.when(sid < NS - 1)
        def _():
            pltpu.sync_copy(zeros_hbm.at[pl.ds(sid * rpt_a, rpt_a)],
                            h_sh.at[pl.ds(sid * rpt_a, rpt_a)])

        @pl.when(sid == NS - 1)
        def _():
            pltpu.sync_copy(zeros_hbm.at[pl.ds(rpt_a * (NS - 1), rpt_z)],
                            h_sh.at[pl.ds(rpt_a * (NS - 1), rpt_z)])

        cp_g.wait()

        plsc.subcore_barrier()

        def fire_g(c, slot):
            pltpu.async_copy(y_hbm.at[gid.at[pl.ds(c * b, b)]],
                             rows.at[slot], sg[slot])

        def wait_g(c, slot):
            pltpu.make_async_copy(y_hbm.at[gid.at[pl.ds(c * b, b)]],
                                  rows.at[slot], sg[slot]).wait()

        def fire_s(c, k3, k6):
            pltpu.async_copy(rows.at[k3], h_sh.at[didr.at[k6]],
                             ss[k3], add=True)

        def wait_s(k3, k6):
            pltpu.make_async_copy(rows.at[k3], h_sh.at[didr.at[k6]],
                                  ss[k3]).wait()

        def visit(c, j, first, last):
            k3 = j % 3
            wait_g(c, k3)
            wait_d()
            fire_s(c, k3, j)
            if not (first and j == 0):
                wait_s((j + 2) % 3, (j + 5) % 6)
            if not last or j + 4 < 5:
                fire_d(c + 4, (j + 4) % 6)
            if not last or j + 2 < 5:
                fire_g(c + 2, (j + 2) % 3)

        for j in range(4):
            fire_d(j, j)
        fire_g(0, 0)
        fire_g(1, 1)
        for c in range(6):
            visit(c, c, first=True, last=False)

        def step(t, carry):
            for j in range(6):
                visit(6 * t + j, j, first=False, last=False)
            return carry

        lax.fori_loop(1, (nch - 5) // 6, step, 0)
        for j in range(5):
            visit(nch - 5 + j, (nch - 5 + j) % 6, first=False, last=True)
        wait_s((nch - 1) % 3, (nch - 1) % 6)

        plsc.subcore_barrier()

        @pl.when(sid < NS - 1)
        def _():
            pltpu.sync_copy(h_sh.at[pl.ds(sid * rpt_a, rpt_a)],
                            out_hbm.at[cid, pl.ds(sid * rpt_a, rpt_a)])

        @pl.when(sid == NS - 1)
        def _():
            pltpu.sync_copy(h_sh.at[pl.ds(rpt_a * (NS - 1), rpt_z)],
                            out_hbm.at[cid, pl.ds(rpt_a * (NS - 1), rpt_z)])

    return sc_scatter, ept, ept_p


def _pad_per_tile(a, ept, ept_p, fill):
    if ept_p == ept:
        return a
    pad = jnp.broadcast_to(jnp.asarray(fill, a.dtype).reshape(-1, 1),
                           (NW, ept_p - ept))
    return jnp.concatenate([a.reshape(NW, ept), pad], axis=1).reshape(-1)


def kernel(x, edge_index, rel_type, weight):
    n, _ = x.shape
    r, _, d_out = weight.shape
    e = edge_index.shape[1]
    y, g, dst = _relation_gemm_gid(x, weight, edge_index, rel_type, 1000)
    sc, ept, ept_p = _make_sc_scatter(n, d_out, e)
    g_p = _pad_per_tile(g, ept, ept_p, 0)
    dst_p = _pad_per_tile(dst, ept, ept_p, n + jnp.arange(NW) % NS)
    zeros = jnp.zeros((n, d_out), jnp.float32)
    partials = sc(y, g_p, dst_p, zeros)
    return _pair_add(partials, 2000)


# speedup vs baseline: 2.0336x; 1.0327x over previous
import functools

import jax
import jax.numpy as jnp
from jax import lax
from jax.experimental import pallas as pl
from jax.experimental.pallas import tpu as pltpu
from jax.experimental.pallas import tpu_sc as plsc

NC = 2
NS = 16
NW = NC * NS


def _gemm_gid_body(n_nodes, x_ref, w_ref, s_ref, r_ref, o_ref, g_ref):
    o_ref[...] = jnp.dot(x_ref[...], w_ref[0],
                         preferred_element_type=jnp.float32)

    @pl.when(jnp.logical_and(pl.program_id(0) == 0, pl.program_id(1) == 0))
    def _():
        g_ref[...] = r_ref[...] * n_nodes + s_ref[...]


def _relation_gemm_gid(x, weight, src, rel, bn):
    n, d_in = x.shape
    r, _, d_out = weight.shape
    e = src.shape[0]
    nbn = n // bn
    return pl.pallas_call(
        functools.partial(_gemm_gid_body, n),
        grid=(nbn, r),
        in_specs=[
            pl.BlockSpec((bn, d_in), lambda i, j: (i, 0)),
            pl.BlockSpec((1, d_in, d_out), lambda i, j: (j, 0, 0)),
            pl.BlockSpec((e,), lambda i, j: (0,)),
            pl.BlockSpec((e,), lambda i, j: (0,)),
        ],
        out_specs=[
            pl.BlockSpec((bn, d_out), lambda i, j, _nbn=nbn: (j * _nbn + i, 0)),
            pl.BlockSpec((e,), lambda i, j: (0,)),
        ],
        out_shape=[
            jax.ShapeDtypeStruct((r * n, d_out), jnp.float32),
            jax.ShapeDtypeStruct((e,), jnp.int32),
        ],
    )(x, weight, src, rel)


def _add_body(p_ref, o_ref):
    o_ref[...] = p_ref[0] + p_ref[1]


def _pair_add(p, bn):
    _, n, d = p.shape
    return pl.pallas_call(
        _add_body,
        grid=(n // bn,),
        in_specs=[pl.BlockSpec((2, bn, d), lambda i: (0, i, 0))],
        out_specs=pl.BlockSpec((bn, d), lambda i: (i, 0)),
        out_shape=jax.ShapeDtypeStruct((n, d), jnp.float32),
    )(p)


def _make_sc_scatter(n_nodes, d, n_edges):
    ept = n_edges // NW
    b = 80
    nch = -(-ept // b)
    nch += (5 - nch) % 6
    ept_p = nch * b
    rpt_a = (n_nodes // NS) & ~7
    rpt_z = n_nodes - rpt_a * (NS - 1)
    assert ept * NW == n_edges and nch % 6 == 5 and nch >= 17
    assert b % 8 == 0 and ept_p % 8 == 0 and rpt_a % 8 == 0

    mesh = plsc.VectorSubcoreMesh(core_axis_name="c", subcore_axis_name="s",
                                  num_cores=NC, num_subcores=NS)

    @functools.partial(
        pl.kernel,
        out_type=jax.ShapeDtypeStruct((NC, n_nodes, d), jnp.float32),
        mesh=mesh,
        scratch_types=[
            pltpu.VMEM((ept_p,), jnp.int32),
            pltpu.VMEM((6, b), jnp.int32),
            pltpu.VMEM((3, b, d), jnp.float32),
            pltpu.VMEM_SHARED((n_nodes + NS, d), jnp.float32),
            pltpu.SemaphoreType.DMA,
            pltpu.SemaphoreType.DMA,
            (pltpu.SemaphoreType.DMA,) * 3,
            (pltpu.SemaphoreType.DMA,) * 3,
        ],
    )
    def sc_scatter(y_hbm, g_hbm, dst_hbm, zeros_hbm, out_hbm,
                   gid, didr, rows, h_sh, sem_i, sem_d, sg, ss):
        cid = lax.axis_index("c")
        sid = lax.axis_index("s")
        wid = cid * NS + sid
        base = wid * ept_p

        cp_g = pltpu.async_copy(g_hbm.at[pl.ds(base, ept_p)], gid, sem_i)

        def fire_d(c, slot):
            pltpu.async_copy(dst_hbm.at[pl.ds(base + c * b, b)],
                             didr.at[slot], sem_d)

        def wait_d():
            pltpu.make_async_copy(dst_hbm.at[pl.ds(base, b)],
                                  didr.at[0], sem_d).wait()

        @pl.when(sid < NS - 1)
        def _():
            pltpu.sync_copy(zeros_hbm.at[pl.ds(sid * rpt_a, rpt_a)],
                            h_sh.at[pl.ds(sid * rpt_a, rpt_a)])

        @pl.when(sid == NS - 1)
        def _():
            pltpu.sync_copy(zeros_hbm.at[pl.ds(rpt_a * (NS - 1), rpt_z)],
                            h_sh.at[pl.ds(rpt_a * (NS - 1), rpt_z)])

        cp_g.wait()

        plsc.subcore_barrier()

        def fire_g(c, slot):
            pltpu.async_copy(y_hbm.at[gid.at[pl.ds(c * b, b)]],
                             rows.at[slot], sg[slot])

        def wait_g(c, slot):
            pltpu.make_async_copy(y_hbm.at[gid.at[pl.ds(c * b, b)]],
                                  rows.at[slot], sg[slot]).wait()

        def fire_s(c, k3, k6):
            pltpu.async_copy(rows.at[k3], h_sh.at[didr.at[k6]],
                             ss[k3], add=True)

        def wait_s(k3, k6):
            pltpu.make_async_copy(rows.at[k3], h_sh.at[didr.at[k6]],
                                  ss[k3]).wait()

        def visit(c, j, first, last):
            k3 = j % 3
            wait_g(c, k3)
            wait_d()
            fire_s(c, k3, j)
            if not (first and j == 0):
                wait_s((j + 2) % 3, (j + 5) % 6)
            if not last or j + 4 < 5:
                fire_d(c + 4, (j + 4) % 6)
            if not last or j + 2 < 5:
                fire_g(c + 2, (j + 2) % 3)

        for j in range(4):
            fire_d(j, j)
        fire_g(0, 0)
        fire_g(1, 1)
        for c in range(6):
            visit(c, c, first=True, last=False)

        def step(t, carry):
            for j in range(6):
                visit(6 * t + j, j, first=False, last=False)
            return carry

        lax.fori_loop(1, (nch - 5) // 6, step, 0)
        for j in range(5):
            visit(nch - 5 + j, (nch - 5 + j) % 6, first=False, last=True)
        wait_s((nch - 1) % 3, (nch - 1) % 6)

        plsc.subcore_barrier()

        @pl.when(sid < NS - 1)
        def _():
            pltpu.sync_copy(h_sh.at[pl.ds(sid * rpt_a, rpt_a)],
                            out_hbm.at[cid, pl.ds(sid * rpt_a, rpt_a)])

        @pl.when(sid == NS - 1)
        def _():
            pltpu.sync_copy(h_sh.at[pl.ds(rpt_a * (NS - 1), rpt_z)],
                            out_hbm.at[cid, pl.ds(rpt_a * (NS - 1), rpt_z)])

    return sc_scatter, ept, ept_p


def _pad_per_tile(a, ept, ept_p, fill):
    if ept_p == ept:
        return a
    pad = jnp.broadcast_to(jnp.asarray(fill, a.dtype).reshape(-1, 1),
                           (NW, ept_p - ept))
    return jnp.concatenate([a.reshape(NW, ept), pad], axis=1).reshape(-1)


def kernel(x, edge_index, rel_type, weight):
    n, _ = x.shape
    r, _, d_out = weight.shape
    e = edge_index.shape[1]
    src = edge_index[0]
    dst = edge_index[1]
    y, g = _relation_gemm_gid(x, weight, src, rel_type, 2000)
    sc, ept, ept_p = _make_sc_scatter(n, d_out, e)
    g_p = _pad_per_tile(g, ept, ept_p, 0)
    dst_p = _pad_per_tile(dst, ept, ept_p, n + jnp.arange(NW) % NS)
    zeros = jnp.zeros((n, d_out), jnp.float32)
    partials = sc(y, g_p, dst_p, zeros)
    return _pair_add(partials, 2000)


# speedup vs baseline: 2.1320x; 1.0484x over previous
import functools

import jax
import jax.numpy as jnp
from jax import lax
from jax.experimental import pallas as pl
from jax.experimental.pallas import tpu as pltpu
from jax.experimental.pallas import tpu_sc as plsc

NC = 2
NS = 16
NW = NC * NS


def _gemm_gid_body(n_nodes, x_ref, w_ref, s_ref, r_ref, o_ref, g_ref):
    o_ref[...] = jnp.dot(x_ref[...], w_ref[0],
                         preferred_element_type=jnp.float32)

    @pl.when(jnp.logical_and(pl.program_id(0) == 0, pl.program_id(1) == 0))
    def _():
        g_ref[...] = r_ref[...] * n_nodes + s_ref[...]


def _relation_gemm_gid(x, weight, src, rel, bn):
    n, d_in = x.shape
    r, _, d_out = weight.shape
    e = src.shape[0]
    nbn = n // bn
    return pl.pallas_call(
        functools.partial(_gemm_gid_body, n),
        grid=(nbn, r),
        in_specs=[
            pl.BlockSpec((bn, d_in), lambda i, j: (i, 0)),
            pl.BlockSpec((1, d_in, d_out), lambda i, j: (j, 0, 0)),
            pl.BlockSpec((e,), lambda i, j: (0,)),
            pl.BlockSpec((e,), lambda i, j: (0,)),
        ],
        out_specs=[
            pl.BlockSpec((bn, d_out), lambda i, j, _nbn=nbn: (j * _nbn + i, 0)),
            pl.BlockSpec((e,), lambda i, j: (0,)),
        ],
        out_shape=[
            jax.ShapeDtypeStruct((r * n, d_out), jnp.float32),
            jax.ShapeDtypeStruct((e,), jnp.int32),
        ],
    )(x, weight, src, rel)


def _add_body(p_ref, o_ref):
    o_ref[...] = p_ref[0] + p_ref[1]


def _pair_add(p, bn):
    _, n, d = p.shape
    return pl.pallas_call(
        _add_body,
        grid=(n // bn,),
        in_specs=[pl.BlockSpec((2, bn, d), lambda i: (0, i, 0))],
        out_specs=pl.BlockSpec((bn, d), lambda i: (i, 0)),
        out_shape=jax.ShapeDtypeStruct((n, d), jnp.float32),
    )(p)


def _make_sc_scatter(n_nodes, d, n_edges):
    ept = n_edges // NW
    b = 80
    nch = -(-ept // b)
    nch += (5 - nch) % 6
    ept_p = nch * b
    rpt_a = (n_nodes // NS) & ~7
    rpt_z = n_nodes - rpt_a * (NS - 1)
    assert ept * NW == n_edges and nch % 6 == 5 and nch >= 17
    assert b % 8 == 0 and ept_p % 8 == 0 and rpt_a % 8 == 0

    mesh = plsc.VectorSubcoreMesh(core_axis_name="c", subcore_axis_name="s",
                                  num_cores=NC, num_subcores=NS)

    @functools.partial(
        pl.kernel,
        out_type=jax.ShapeDtypeStruct((NC, n_nodes, d), jnp.float32),
        mesh=mesh,
        scratch_types=[
            pltpu.VMEM((ept_p,), jnp.int32),
            pltpu.VMEM((6, b), jnp.int32),
            pltpu.VMEM((3, b, d), jnp.float32),
            pltpu.VMEM_SHARED((n_nodes + NS, d), jnp.float32),
            pltpu.SemaphoreType.DMA,
            pltpu.SemaphoreType.DMA,
            (pltpu.SemaphoreType.DMA,) * 3,
            (pltpu.SemaphoreType.DMA,) * 3,
        ],
    )
    def sc_scatter(y_hbm, g_hbm, dst_hbm, zeros_hbm, out_hbm,
                   gid, didr, rows, h_sh, sem_i, sem_d, sg, ss):
        cid = lax.axis_index("c")
        sid = lax.axis_index("s")
        wid = cid * NS + sid
        base = wid * ept_p

        cp_g = pltpu.async_copy(g_hbm.at[pl.ds(base, ept_p)], gid, sem_i)

        def fire_d(c, slot):
            pltpu.async_copy(dst_hbm.at[pl.ds(base + c * b, b)],
                             didr.at[slot], sem_d)

        def wait_d():
            pltpu.make_async_copy(dst_hbm.at[pl.ds(base, b)],
                                  didr.at[0], sem_d).wait()

        @pl.when(sid < NS - 1)
        def _():
            pltpu.sync_copy(zeros_hbm.at[pl.ds(sid * rpt_a, rpt_a)],
                            h_sh.at[pl.ds(sid * rpt_a, rpt_a)])

        @pl.when(sid == NS - 1)
        def _():
            pltpu.sync_copy(zeros_hbm.at[pl.ds(rpt_a * (NS - 1), rpt_z)],
                            h_sh.at[pl.ds(rpt_a * (NS - 1), rpt_z)])

        cp_g.wait()

        plsc.subcore_barrier()

        def fire_g(c, slot):
            pltpu.async_copy(y_hbm.at[gid.at[pl.ds(c * b, b)]],
                             rows.at[slot], sg[slot])

        def wait_g(c, slot):
            pltpu.make_async_copy(y_hbm.at[gid.at[pl.ds(c * b, b)]],
                                  rows.at[slot], sg[slot]).wait()

        def fire_s(c, k3, k6):
            pltpu.async_copy(rows.at[k3], h_sh.at[didr.at[k6]],
                             ss[k3], add=True)

        def wait_s(k3, k6):
            pltpu.make_async_copy(rows.at[k3], h_sh.at[didr.at[k6]],
                                  ss[k3]).wait()

        def visit(c, j, first, last):
            k3 = j % 3
            wait_g(c, k3)
            wait_d()
            fire_s(c, k3, j)
            if not (first and j == 0):
                wait_s((j + 2) % 3, (j + 5) % 6)
            if not last or j + 4 < 5:
                fire_d(c + 4, (j + 4) % 6)
            if not last or j + 2 < 5:
                fire_g(c + 2, (j + 2) % 3)

        for j in range(4):
            fire_d(j, j)
        fire_g(0, 0)
        fire_g(1, 1)
        for c in range(6):
            visit(c, c, first=True, last=False)

        def step(t, carry):
            for j in range(6):
                visit(6 * t + j, j, first=False, last=False)
            return carry

        lax.fori_loop(1, (nch - 5) // 6, step, 0)
        for j in range(5):
            visit(nch - 5 + j, (nch - 5 + j) % 6, first=False, last=True)
        wait_s((nch - 1) % 3, (nch - 1) % 6)

        plsc.subcore_barrier()

        @pl.when(sid < NS - 1)
        def _():
            pltpu.sync_copy(h_sh.at[pl.ds(sid * rpt_a, rpt_a)],
                            out_hbm.at[cid, pl.ds(sid * rpt_a, rpt_a)])

        @pl.when(sid == NS - 1)
        def _():
            pltpu.sync_copy(h_sh.at[pl.ds(rpt_a * (NS - 1), rpt_z)],
                            out_hbm.at[cid, pl.ds(rpt_a * (NS - 1), rpt_z)])

    return sc_scatter, ept, ept_p


def _pad_per_tile(a, ept, ept_p, fill):
    if ept_p == ept:
        return a
    pad = jnp.broadcast_to(jnp.asarray(fill, a.dtype).reshape(-1, 1),
                           (NW, ept_p - ept))
    return jnp.concatenate([a.reshape(NW, ept), pad], axis=1).reshape(-1)


def kernel(x, edge_index, rel_type, weight):
    n, _ = x.shape
    r, _, d_out = weight.shape
    e = edge_index.shape[1]
    src = edge_index[0]
    dst = edge_index[1]
    y, g = _relation_gemm_gid(x, weight, src, rel_type, 5000)
    sc, ept, ept_p = _make_sc_scatter(n, d_out, e)
    g_p = _pad_per_tile(g, ept, ept_p, 0)
    dst_p = _pad_per_tile(dst, ept, ept_p, n + jnp.arange(NW) % NS)
    zeros = jnp.zeros((n, d_out), jnp.float32)
    partials = sc(y, g_p, dst_p, zeros)
    return _pair_add(partials, 2000)


# speedup vs baseline: 2.1984x; 1.0311x over previous
import functools

import jax
import jax.numpy as jnp
from jax import lax
from jax.experimental import pallas as pl
from jax.experimental.pallas import tpu as pltpu
from jax.experimental.pallas import tpu_sc as plsc

NC = 2
NS = 16
NW = NC * NS


def _gemm_gid_body(n_nodes, x_ref, w_ref, s_ref, r_ref, o_ref, g_ref):
    o_ref[...] = jnp.dot(x_ref[...], w_ref[0],
                         preferred_element_type=jnp.float32)

    @pl.when(jnp.logical_and(pl.program_id(0) == 0, pl.program_id(1) == 0))
    def _():
        g_ref[...] = r_ref[...] * n_nodes + s_ref[...]


def _relation_gemm_gid(x, weight, src, rel, bn):
    n, d_in = x.shape
    r, _, d_out = weight.shape
    e = src.shape[0]
    nbn = n // bn
    return pl.pallas_call(
        functools.partial(_gemm_gid_body, n),
        grid=(nbn, r),
        in_specs=[
            pl.BlockSpec((bn, d_in), lambda i, j: (i, 0)),
            pl.BlockSpec((1, d_in, d_out), lambda i, j: (j, 0, 0)),
            pl.BlockSpec((e,), lambda i, j: (0,)),
            pl.BlockSpec((e,), lambda i, j: (0,)),
        ],
        out_specs=[
            pl.BlockSpec((bn, d_out), lambda i, j, _nbn=nbn: (j * _nbn + i, 0)),
            pl.BlockSpec((e,), lambda i, j: (0,)),
        ],
        out_shape=[
            jax.ShapeDtypeStruct((r * n, d_out), jnp.float32),
            jax.ShapeDtypeStruct((e,), jnp.int32),
        ],
    )(x, weight, src, rel)


def _add_body(p_ref, o_ref):
    o_ref[...] = p_ref[0] + p_ref[1]


def _pair_add(p, bn):
    _, n, d = p.shape
    return pl.pallas_call(
        _add_body,
        grid=(n // bn,),
        in_specs=[pl.BlockSpec((2, bn, d), lambda i: (0, i, 0))],
        out_specs=pl.BlockSpec((bn, d), lambda i: (i, 0)),
        out_shape=jax.ShapeDtypeStruct((n, d), jnp.float32),
    )(p)


def _make_sc_scatter(n_nodes, d, n_edges):
    ept = n_edges // NW
    b = 80
    nch = -(-ept // b)
    nch += (5 - nch) % 6
    ept_p = nch * b
    rpt_a = (n_nodes // NS) & ~7
    rpt_z = n_nodes - rpt_a * (NS - 1)
    assert ept * NW == n_edges and nch % 6 == 5 and nch >= 17
    assert b % 8 == 0 and ept_p % 8 == 0 and rpt_a % 8 == 0

    mesh = plsc.VectorSubcoreMesh(core_axis_name="c", subcore_axis_name="s",
                                  num_cores=NC, num_subcores=NS)

    @functools.partial(
        pl.kernel,
        out_type=jax.ShapeDtypeStruct((NC, n_nodes, d), jnp.float32),
        mesh=mesh,
        scratch_types=[
            pltpu.VMEM((ept_p,), jnp.int32),
            pltpu.VMEM((6, b), jnp.int32),
            pltpu.VMEM((3, b, d), jnp.float32),
            pltpu.VMEM_SHARED((n_nodes + NS, d), jnp.float32),
            pltpu.SemaphoreType.DMA,
            pltpu.SemaphoreType.DMA,
            (pltpu.SemaphoreType.DMA,) * 3,
            (pltpu.SemaphoreType.DMA,) * 3,
        ],
    )
    def sc_scatter(y_hbm, g_hbm, dst_hbm, zeros_hbm, out_hbm,
                   gid, didr, rows, h_sh, sem_i, sem_d, sg, ss):
        cid = lax.axis_index("c")
        sid = lax.axis_index("s")
        wid = cid * NS + sid
        base = wid * ept_p

        cp_g = pltpu.async_copy(g_hbm.at[pl.ds(base, ept_p)], gid, sem_i)

        def fire_d(c, slot):
            pltpu.async_copy(dst_hbm.at[pl.ds(base + c * b, b)],
                             didr.at[slot], sem_d)

        def wait_d():
            pltpu.make_async_copy(dst_hbm.at[pl.ds(base, b)],
                                  didr.at[0], sem_d).wait()

        @pl.when(sid < NS - 1)
        def _():
            pltpu.sync_copy(zeros_hbm.at[pl.ds(sid * rpt_a, rpt_a)],
                            h_sh.at[pl.ds(sid * rpt_a, rpt_a)])

        @pl.when(sid == NS - 1)
        def _():
            pltpu.sync_copy(zeros_hbm.at[pl.ds(rpt_a * (NS - 1), rpt_z)],
                            h_sh.at[pl.ds(rpt_a * (NS - 1), rpt_z)])

        cp_g.wait()

        plsc.subcore_barrier()

        def fire_g(c, slot):
            pltpu.async_copy(y_hbm.at[gid.at[pl.ds(c * b, b)]],
                             rows.at[slot], sg[slot])

        def wait_g(c, slot):
            pltpu.make_async_copy(y_hbm.at[gid.at[pl.ds(c * b, b)]],
                                  rows.at[slot], sg[slot]).wait()

        def fire_s(c, k3, k6):
            pltpu.async_copy(rows.at[k3], h_sh.at[didr.at[k6]],
                             ss[k3], add=True)

        def wait_s(k3, k6):
            pltpu.make_async_copy(rows.at[k3], h_sh.at[didr.at[k6]],
                                  ss[k3]).wait()

        def visit(c, j, first, last):
            k3 = j % 3
            wait_g(c, k3)
            wait_d()
            fire_s(c, k3, j)
            if not (first and j == 0):
                wait_s((j + 2) % 3, (j + 5) % 6)
            if not last or j + 4 < 5:
                fire_d(c + 4, (j + 4) % 6)
            if not last or j + 2 < 5:
                fire_g(c + 2, (j + 2) % 3)

        for j in range(4):
            fire_d(j, j)
        fire_g(0, 0)
        fire_g(1, 1)
        for c in range(6):
            visit(c, c, first=True, last=False)

        def step(t, carry):
            for j in range(6):
                visit(6 * t + j, j, first=False, last=False)
            return carry

        lax.fori_loop(1, (nch - 5) // 6, step, 0)
        for j in range(5):
            visit(nch - 5 + j, (nch - 5 + j) % 6, first=False, last=True)
        wait_s((nch - 1) % 3, (nch - 1) % 6)

        plsc.subcore_barrier()

        @pl.when(sid < NS - 1)
        def _():
            pltpu.sync_copy(h_sh.at[pl.ds(sid * rpt_a, rpt_a)],
                            out_hbm.at[cid, pl.ds(sid * rpt_a, rpt_a)])

        @pl.when(sid == NS - 1)
        def _():
            pltpu.sync_copy(h_sh.at[pl.ds(rpt_a * (NS - 1), rpt_z)],
                            out_hbm.at[cid, pl.ds(rpt_a * (NS - 1), rpt_z)])

    return sc_scatter, ept, ept_p


def _pad_per_tile(a, ept, ept_p, fill):
    if ept_p == ept:
        return a
    pad = jnp.broadcast_to(jnp.asarray(fill, a.dtype).reshape(-1, 1),
                           (NW, ept_p - ept))
    return jnp.concatenate([a.reshape(NW, ept), pad], axis=1).reshape(-1)


def kernel(x, edge_index, rel_type, weight):
    n, _ = x.shape
    r, _, d_out = weight.shape
    e = edge_index.shape[1]
    src = edge_index[0]
    dst = edge_index[1]
    y, g = _relation_gemm_gid(x, weight, src, rel_type, 10000)
    sc, ept, ept_p = _make_sc_scatter(n, d_out, e)
    g_p = _pad_per_tile(g, ept, ept_p, 0)
    dst_p = _pad_per_tile(dst, ept, ept_p, n + jnp.arange(NW) % NS)
    zeros = jnp.zeros((n, d_out), jnp.float32)
    partials = sc(y, g_p, dst_p, zeros)
    return _pair_add(partials, 2000)
